# baseline (speedup 1.0000x reference)
import functools

import jax
import jax.numpy as jnp
from jax import lax
from jax.experimental import pallas as pl
from jax.experimental.pallas import tpu as pltpu
from jax.experimental.pallas import tpu_sc as plsc

_E = 8
_K = 2
_DIN = 768
_DH = 3072
_DOUT = 768
_N = 2048

_TM = 128
_NPAIR = _N * _K
_P = 5120
_G = _P // _TM

_NC = 2
_NS = 16
_NW = _NC * _NS
_CP = _NPAIR // _NW
_CT = _N // _NW


def _router_kernel(x_ref, gw_ref, gb_ref,
                   probs_ref, aux_ref, twa_ref, twb_ref,
                   pa_ref, pb_ref, apos_ref, eot_ref):
    x = x_ref[:]
    logits = jnp.dot(x, gw_ref[:], preferred_element_type=jnp.float32) + gb_ref[:]
    m = jnp.max(logits, axis=1, keepdims=True)
    ex = jnp.exp(logits - m)
    probs = ex / jnp.sum(ex, axis=1, keepdims=True)
    probs_ref[:] = probs

    mp = jnp.mean(probs, axis=0, keepdims=True)
    aux_ref[:] = jnp.sum(mp * jnp.log(mp * _E + 1e-10), axis=1, keepdims=True)

    ii = lax.broadcasted_iota(jnp.int32, (_N, _E), 1)
    v1 = jnp.max(probs, axis=1, keepdims=True)
    i1 = jnp.min(jnp.where(probs >= v1, ii, _E), axis=1, keepdims=True)
    oh1 = ii == i1
    pm = jnp.where(oh1, -1.0, probs)
    v2 = jnp.max(pm, axis=1, keepdims=True)
    i2 = jnp.min(jnp.where(pm >= v2, ii, _E), axis=1, keepdims=True)
    oh2 = ii == i2
    den = v1 + v2 + 1e-10
    wa = v1 / den
    wb = v2 / den
    twa_ref[:] = jnp.broadcast_to(wa, (_N, 16))
    twb_ref[:] = jnp.broadcast_to(wb, (_N, 16))

    cnt = oh1.astype(jnp.float32) + oh2.astype(jnp.float32)
    nb = _N // 128
    r = lax.broadcasted_iota(jnp.int32, (128, 128), 0)
    c = lax.broadcasted_iota(jnp.int32, (128, 128), 1)
    tril = (c < r).astype(jnp.float32)
    blocks, sums = [], []
    for b in range(nb):
        blk = cnt[b * 128:(b + 1) * 128, :]
        blocks.append(jnp.dot(tril, blk, preferred_element_type=jnp.float32))
        sums.append(jnp.sum(blk, axis=0, keepdims=True))
    s = jnp.concatenate(sums, axis=0)
    r2 = lax.broadcasted_iota(jnp.int32, (nb, nb), 0)
    c2 = lax.broadcasted_iota(jnp.int32, (nb, nb), 1)
    tril2 = (c2 < r2).astype(jnp.float32)
    carry = jnp.dot(tril2, s, preferred_element_type=jnp.float32)
    cex = jnp.concatenate(
        [blocks[b] + carry[b:b + 1, :] for b in range(nb)], axis=0)

    tot = jnp.sum(s, axis=0, keepdims=True)
    cpad = (tot.astype(jnp.int32) + (_TM - 1)) // _TM * _TM
    r3 = lax.broadcasted_iota(jnp.int32, (_E, _E), 0)
    c3 = lax.broadcasted_iota(jnp.int32, (_E, _E), 1)
    sup = (r3 < c3).astype(jnp.float32)
    off = jnp.dot(cpad.astype(jnp.float32), sup,
                  preferred_element_type=jnp.float32)

    base = off + cex
    pa = jnp.sum(jnp.where(oh1, base, 0.0), axis=1, keepdims=True)
    pb = jnp.sum(jnp.where(oh2, base, 0.0), axis=1, keepdims=True)
    pa_ref[:] = pa.astype(jnp.int32)
    pb_ref[:] = pb.astype(jnp.int32)
    apos_ref[:] = jnp.where(cnt > 0.0, base, -1.0)

    tv = (lax.broadcasted_iota(jnp.int32, (_G, _E), 0) * _TM).astype(jnp.float32)
    eot = jnp.sum((off <= tv).astype(jnp.float32), axis=1, keepdims=True) - 1.0
    eot_ref[:] = eot.astype(jnp.int32)


_router = pl.pallas_call(
    _router_kernel,
    out_shape=[
        jax.ShapeDtypeStruct((_N, _E), jnp.float32),
        jax.ShapeDtypeStruct((1, 1), jnp.float32),
        jax.ShapeDtypeStruct((_N, 16), jnp.float32),
        jax.ShapeDtypeStruct((_N, 16), jnp.float32),
        jax.ShapeDtypeStruct((_N, 1), jnp.int32),
        jax.ShapeDtypeStruct((_N, 1), jnp.int32),
        jax.ShapeDtypeStruct((_N, _E), jnp.float32),
        jax.ShapeDtypeStruct((_G, 1), jnp.int32),
    ],
)


@functools.cache
def _sc_mesh():
    return plsc.VectorSubcoreMesh(core_axis_name="c", subcore_axis_name="s")


def _ffn_kernel(eot_ref, apos_ref, x_ref, w1_ref, b1_ref, w2_ref, b2_ref,
                o_ref):
    t = pl.program_id(0)
    e = eot_ref[t]
    ii = lax.broadcasted_iota(jnp.int32, (_N, _E), 1)
    acol = jnp.max(jnp.where(ii == e, apos_ref[:], -1.0), axis=1,
                   keepdims=True)
    g = (lax.broadcasted_iota(jnp.int32, (1, _TM), 1) + t * _TM).astype(
        jnp.float32)
    s = (acol == g).astype(jnp.float32)
    xt = lax.dot_general(s, x_ref[:], (((0,), (0,)), ((), ())),
                         preferred_element_type=jnp.float32)
    h = jnp.dot(xt, w1_ref[0], preferred_element_type=jnp.float32)
    h = jnp.maximum(h + b1_ref[0], 0.0)
    o = jnp.dot(h, w2_ref[0], preferred_element_type=jnp.float32) + b2_ref[0]
    o_ref[:] = o


_ffn = pl.pallas_call(
    _ffn_kernel,
    grid_spec=pltpu.PrefetchScalarGridSpec(
        num_scalar_prefetch=1,
        grid=(_G,),
        in_specs=[
            pl.BlockSpec((_N, _E), lambda t, eot: (0, 0)),
            pl.BlockSpec((_N, _DIN), lambda t, eot: (0, 0)),
            pl.BlockSpec((1, _DIN, _DH), lambda t, eot: (eot[t], 0, 0)),
            pl.BlockSpec((1, 1, _DH), lambda t, eot: (eot[t], 0, 0)),
            pl.BlockSpec((1, _DH, _DOUT), lambda t, eot: (eot[t], 0, 0)),
            pl.BlockSpec((1, 1, _DOUT), lambda t, eot: (eot[t], 0, 0)),
        ],
        out_specs=pl.BlockSpec((_TM, _DOUT), lambda t, eot: (t, 0)),
    ),
    out_shape=jax.ShapeDtypeStruct((_P, _DOUT), jnp.float32),
    compiler_params=pltpu.CompilerParams(
        dimension_semantics=("arbitrary",)),
)


def _combine_body(p_hbm, pa_hbm, pb_hbm, twa_hbm, twb_hbm, out_hbm,
                  pa_v, pb_v, twa_v, twb_v, a_v, b_v, sem1, sem2):
    wid = lax.axis_index("s") * _NC + lax.axis_index("c")
    bt = wid * _CT
    pltpu.sync_copy(pa_hbm.at[pl.ds(bt, _CT)], pa_v)
    pltpu.sync_copy(pb_hbm.at[pl.ds(bt, _CT)], pb_v)
    pltpu.sync_copy(twa_hbm.at[pl.ds(bt, _CT)], twa_v)
    pltpu.sync_copy(twb_hbm.at[pl.ds(bt, _CT)], twb_v)
    ca = pltpu.async_copy(p_hbm.at[pa_v], a_v, sem1)
    cb = pltpu.async_copy(p_hbm.at[pb_v], b_v, sem2)
    ca.wait()
    cb.wait()

    def body(t, carry):
        wa = twa_v[t, :]
        wb = twb_v[t, :]
        for c in range(_DOUT // 16):
            sl = pl.ds(c * 16, 16)
            a_v[t, sl] = wa * a_v[t, sl] + wb * b_v[t, sl]
        return carry

    lax.fori_loop(0, _CT, body, 0)
    pltpu.sync_copy(a_v, out_hbm.at[pl.ds(bt, _CT)])


@functools.cache
def _combine():
    return pl.kernel(
        _combine_body,
        out_type=jax.ShapeDtypeStruct((_N, _DOUT), jnp.float32),
        mesh=_sc_mesh(),
        scratch_types=[
            pltpu.VMEM((_CT,), jnp.int32),
            pltpu.VMEM((_CT,), jnp.int32),
            pltpu.VMEM((_CT, 16), jnp.float32),
            pltpu.VMEM((_CT, 16), jnp.float32),
            pltpu.VMEM((_CT, _DOUT), jnp.float32),
            pltpu.VMEM((_CT, _DOUT), jnp.float32),
            pltpu.SemaphoreType.DMA,
            pltpu.SemaphoreType.DMA,
        ],
    )


def kernel(x, gate_w, gate_b, W1, b1, W2, b2):
    probs, aux, twa, twb, pa2, pb2, apos, eot2 = _router(
        x, gate_w, gate_b.reshape(1, _E))
    pos_a = pa2.reshape(_N)
    pos_b = pb2.reshape(_N)
    eot = eot2.reshape(_G)
    pairs_out = _ffn(eot, apos, x, W1, b1.reshape(_E, 1, _DH),
                     W2, b2.reshape(_E, 1, _DOUT))
    out = _combine()(pairs_out, pos_a, pos_b, twa, twb)
    return out, aux.reshape(()), probs


# speedup vs baseline: 1.0826x; 1.0826x over previous
import functools

import jax
import jax.numpy as jnp
from jax import lax
from jax.experimental import pallas as pl
from jax.experimental.pallas import tpu as pltpu
from jax.experimental.pallas import tpu_sc as plsc

_E = 8
_K = 2
_DIN = 768
_DH = 3072
_DOUT = 768
_N = 2048

_TM = 128
_NPAIR = _N * _K
_P = 5120
_G = _P // _TM

_NC = 2
_NS = 16
_NW = _NC * _NS
_CP = _NPAIR // _NW
_CT = _N // _NW


def _router_kernel(x_ref, gw_ref, gb_ref,
                   probs_ref, aux_ref, twa_ref, twb_ref,
                   pa_ref, pb_ref, apos_ref, eot_ref):
    x = x_ref[:]
    logits = jnp.dot(x, gw_ref[:], preferred_element_type=jnp.float32) + gb_ref[:]
    m = jnp.max(logits, axis=1, keepdims=True)
    ex = jnp.exp(logits - m)
    probs = ex / jnp.sum(ex, axis=1, keepdims=True)
    probs_ref[:] = probs

    mp = jnp.mean(probs, axis=0, keepdims=True)
    aux_ref[:] = jnp.sum(mp * jnp.log(mp * _E + 1e-10), axis=1, keepdims=True)

    ii = lax.broadcasted_iota(jnp.int32, (_N, _E), 1)
    v1 = jnp.max(probs, axis=1, keepdims=True)
    i1 = jnp.min(jnp.where(probs >= v1, ii, _E), axis=1, keepdims=True)
    oh1 = ii == i1
    pm = jnp.where(oh1, -1.0, probs)
    v2 = jnp.max(pm, axis=1, keepdims=True)
    i2 = jnp.min(jnp.where(pm >= v2, ii, _E), axis=1, keepdims=True)
    oh2 = ii == i2
    den = v1 + v2 + 1e-10
    wa = v1 / den
    wb = v2 / den
    twa_ref[:] = jnp.broadcast_to(wa, (_N, 16))
    twb_ref[:] = jnp.broadcast_to(wb, (_N, 16))

    cnt = oh1.astype(jnp.float32) + oh2.astype(jnp.float32)
    nb = _N // 128
    r = lax.broadcasted_iota(jnp.int32, (128, 128), 0)
    c = lax.broadcasted_iota(jnp.int32, (128, 128), 1)
    tril = (c < r).astype(jnp.float32)
    blocks, sums = [], []
    for b in range(nb):
        blk = cnt[b * 128:(b + 1) * 128, :]
        blocks.append(jnp.dot(tril, blk, preferred_element_type=jnp.float32))
        sums.append(jnp.sum(blk, axis=0, keepdims=True))
    s = jnp.concatenate(sums, axis=0)
    r2 = lax.broadcasted_iota(jnp.int32, (nb, nb), 0)
    c2 = lax.broadcasted_iota(jnp.int32, (nb, nb), 1)
    tril2 = (c2 < r2).astype(jnp.float32)
    carry = jnp.dot(tril2, s, preferred_element_type=jnp.float32)
    cex = jnp.concatenate(
        [blocks[b] + carry[b:b + 1, :] for b in range(nb)], axis=0)

    tot = jnp.sum(s, axis=0, keepdims=True)
    cpad = (tot.astype(jnp.int32) + (_TM - 1)) // _TM * _TM
    r3 = lax.broadcasted_iota(jnp.int32, (_E, _E), 0)
    c3 = lax.broadcasted_iota(jnp.int32, (_E, _E), 1)
    sup = (r3 < c3).astype(jnp.float32)
    off = jnp.dot(cpad.astype(jnp.float32), sup,
                  preferred_element_type=jnp.float32)

    base = off + cex
    pa = jnp.sum(jnp.where(oh1, base, 0.0), axis=1, keepdims=True)
    pb = jnp.sum(jnp.where(oh2, base, 0.0), axis=1, keepdims=True)
    pa_ref[:] = pa.astype(jnp.int32)
    pb_ref[:] = pb.astype(jnp.int32)
    apos_ref[:] = jnp.transpose(jnp.where(cnt > 0.0, base, -1.0))

    tv = (lax.broadcasted_iota(jnp.int32, (_G, _E), 0) * _TM).astype(jnp.float32)
    eot = jnp.sum((off <= tv).astype(jnp.float32), axis=1, keepdims=True) - 1.0
    eot_ref[:] = eot.astype(jnp.int32)


_router = pl.pallas_call(
    _router_kernel,
    out_shape=[
        jax.ShapeDtypeStruct((_N, _E), jnp.float32),
        jax.ShapeDtypeStruct((1, 1), jnp.float32),
        jax.ShapeDtypeStruct((_N, 16), jnp.float32),
        jax.ShapeDtypeStruct((_N, 16), jnp.float32),
        jax.ShapeDtypeStruct((_N, 1), jnp.int32),
        jax.ShapeDtypeStruct((_N, 1), jnp.int32),
        jax.ShapeDtypeStruct((_E, _N), jnp.float32),
        jax.ShapeDtypeStruct((_G, 1), jnp.int32),
    ],
)


@functools.cache
def _sc_mesh():
    return plsc.VectorSubcoreMesh(core_axis_name="c", subcore_axis_name="s")


def _ffn_kernel(eot_ref, apos_ref, x_ref, w1_ref, b1_ref, w2_ref, b2_ref,
                o_ref):
    t = pl.program_id(0)
    e = eot_ref[t]
    ii = lax.broadcasted_iota(jnp.int32, (_E, _N), 0)
    arow = jnp.max(jnp.where(ii == e, apos_ref[:], -1.0), axis=0,
                   keepdims=True)
    g = (lax.broadcasted_iota(jnp.int32, (_TM, 1), 0) + t * _TM).astype(
        jnp.float32)
    s = (arow == g).astype(jnp.float32)
    xt = jnp.dot(s, x_ref[:], preferred_element_type=jnp.float32)
    h = jnp.dot(xt, w1_ref[0], preferred_element_type=jnp.float32)
    h = jnp.maximum(h + b1_ref[0], 0.0)
    o = jnp.dot(h, w2_ref[0], preferred_element_type=jnp.float32) + b2_ref[0]
    o_ref[:] = o


_ffn = pl.pallas_call(
    _ffn_kernel,
    grid_spec=pltpu.PrefetchScalarGridSpec(
        num_scalar_prefetch=1,
        grid=(_G,),
        in_specs=[
            pl.BlockSpec((_E, _N), lambda t, eot: (0, 0)),
            pl.BlockSpec((_N, _DIN), lambda t, eot: (0, 0)),
            pl.BlockSpec((1, _DIN, _DH), lambda t, eot: (eot[t], 0, 0)),
            pl.BlockSpec((1, 1, _DH), lambda t, eot: (eot[t], 0, 0)),
            pl.BlockSpec((1, _DH, _DOUT), lambda t, eot: (eot[t], 0, 0)),
            pl.BlockSpec((1, 1, _DOUT), lambda t, eot: (eot[t], 0, 0)),
        ],
        out_specs=pl.BlockSpec((_TM, _DOUT), lambda t, eot: (t, 0)),
    ),
    out_shape=jax.ShapeDtypeStruct((_P, _DOUT), jnp.float32),
    compiler_params=pltpu.CompilerParams(
        dimension_semantics=("arbitrary",)),
)


def _combine_body(p_hbm, pa_hbm, pb_hbm, twa_hbm, twb_hbm, out_hbm,
                  pa_v, pb_v, twa_v, twb_v, a_v, b_v, sem1, sem2):
    wid = lax.axis_index("s") * _NC + lax.axis_index("c")
    bt = wid * _CT
    pltpu.sync_copy(pa_hbm.at[pl.ds(bt, _CT)], pa_v)
    pltpu.sync_copy(pb_hbm.at[pl.ds(bt, _CT)], pb_v)
    pltpu.sync_copy(twa_hbm.at[pl.ds(bt, _CT)], twa_v)
    pltpu.sync_copy(twb_hbm.at[pl.ds(bt, _CT)], twb_v)
    ca = pltpu.async_copy(p_hbm.at[pa_v], a_v, sem1)
    cb = pltpu.async_copy(p_hbm.at[pb_v], b_v, sem2)
    ca.wait()
    cb.wait()

    def body(t, carry):
        wa = twa_v[t, :]
        wb = twb_v[t, :]
        for c in range(_DOUT // 16):
            sl = pl.ds(c * 16, 16)
            a_v[t, sl] = wa * a_v[t, sl] + wb * b_v[t, sl]
        return carry

    lax.fori_loop(0, _CT, body, 0)
    pltpu.sync_copy(a_v, out_hbm.at[pl.ds(bt, _CT)])


@functools.cache
def _combine():
    return pl.kernel(
        _combine_body,
        out_type=jax.ShapeDtypeStruct((_N, _DOUT), jnp.float32),
        mesh=_sc_mesh(),
        scratch_types=[
            pltpu.VMEM((_CT,), jnp.int32),
            pltpu.VMEM((_CT,), jnp.int32),
            pltpu.VMEM((_CT, 16), jnp.float32),
            pltpu.VMEM((_CT, 16), jnp.float32),
            pltpu.VMEM((_CT, _DOUT), jnp.float32),
            pltpu.VMEM((_CT, _DOUT), jnp.float32),
            pltpu.SemaphoreType.DMA,
            pltpu.SemaphoreType.DMA,
        ],
    )


def kernel(x, gate_w, gate_b, W1, b1, W2, b2):
    probs, aux, twa, twb, pa2, pb2, apos, eot2 = _router(
        x, gate_w, gate_b.reshape(1, _E))
    pos_a = pa2.reshape(_N)
    pos_b = pb2.reshape(_N)
    eot = eot2.reshape(_G)
    pairs_out = _ffn(eot, apos, x, W1, b1.reshape(_E, 1, _DH),
                     W2, b2.reshape(_E, 1, _DOUT))
    out = _combine()(pairs_out, pos_a, pos_b, twa, twb)
    return out, aux.reshape(()), probs


# speedup vs baseline: 1.1677x; 1.0787x over previous
import functools

import jax
import jax.numpy as jnp
from jax import lax
from jax.experimental import pallas as pl
from jax.experimental.pallas import tpu as pltpu
from jax.experimental.pallas import tpu_sc as plsc

_E = 8
_K = 2
_DIN = 768
_DH = 3072
_DOUT = 768
_N = 2048

_TM = 128
_NPAIR = _N * _K
_P = 5120
_G = _P // _TM
_NB = _N // _TM

_NC = 2
_NS = 16
_NW = _NC * _NS
_CT = _N // _NW


def _router_kernel(x_ref, gw_ref, gb_ref,
                   probs_ref, aux_ref, twa_ref, twb_ref,
                   pa_ref, pb_ref, eot_ref):
    x = x_ref[:]
    logits = jnp.dot(x, gw_ref[:], preferred_element_type=jnp.float32) + gb_ref[:]
    m = jnp.max(logits, axis=1, keepdims=True)
    ex = jnp.exp(logits - m)
    probs = ex / jnp.sum(ex, axis=1, keepdims=True)
    probs_ref[:] = probs

    mp = jnp.mean(probs, axis=0, keepdims=True)
    aux_ref[:] = jnp.sum(mp * jnp.log(mp * _E + 1e-10), axis=1, keepdims=True)

    ii = lax.broadcasted_iota(jnp.int32, (_N, _E), 1)
    v1 = jnp.max(probs, axis=1, keepdims=True)
    i1 = jnp.min(jnp.where(probs >= v1, ii, _E), axis=1, keepdims=True)
    oh1 = ii == i1
    pm = jnp.where(oh1, -1.0, probs)
    v2 = jnp.max(pm, axis=1, keepdims=True)
    i2 = jnp.min(jnp.where(pm >= v2, ii, _E), axis=1, keepdims=True)
    oh2 = ii == i2
    den = v1 + v2 + 1e-10
    twa_ref[:] = jnp.broadcast_to(v1 / den, (_N, 16))
    twb_ref[:] = jnp.broadcast_to(v2 / den, (_N, 16))

    cnt = oh1.astype(jnp.float32) + oh2.astype(jnp.float32)
    nb = _N // 128
    r = lax.broadcasted_iota(jnp.int32, (128, 128), 0)
    c = lax.broadcasted_iota(jnp.int32, (128, 128), 1)
    tril = (c < r).astype(jnp.float32)
    blocks, sums = [], []
    for b in range(nb):
        blk = cnt[b * 128:(b + 1) * 128, :]
        blocks.append(jnp.dot(tril, blk, preferred_element_type=jnp.float32))
        sums.append(jnp.sum(blk, axis=0, keepdims=True))
    s = jnp.concatenate(sums, axis=0)
    r2 = lax.broadcasted_iota(jnp.int32, (nb, nb), 0)
    c2 = lax.broadcasted_iota(jnp.int32, (nb, nb), 1)
    tril2 = (c2 < r2).astype(jnp.float32)
    carry = jnp.dot(tril2, s, preferred_element_type=jnp.float32)
    cex = jnp.concatenate(
        [blocks[b] + carry[b:b + 1, :] for b in range(nb)], axis=0)

    tot = jnp.sum(s, axis=0, keepdims=True)
    cpad = (tot.astype(jnp.int32) + (_TM - 1)) // _TM * _TM
    r3 = lax.broadcasted_iota(jnp.int32, (_E, _E), 0)
    c3 = lax.broadcasted_iota(jnp.int32, (_E, _E), 1)
    sup = (r3 < c3).astype(jnp.float32)
    off = jnp.dot(cpad.astype(jnp.float32), sup,
                  preferred_element_type=jnp.float32)

    base = off + cex
    pa = jnp.sum(jnp.where(oh1, base, 0.0), axis=1, keepdims=True)
    pb = jnp.sum(jnp.where(oh2, base, 0.0), axis=1, keepdims=True)
    pa_ref[:] = pa.astype(jnp.int32)
    pb_ref[:] = pb.astype(jnp.int32)

    tv = (lax.broadcasted_iota(jnp.int32, (_G, _E), 0) * _TM).astype(jnp.float32)
    eot = jnp.sum((off <= tv).astype(jnp.float32), axis=1, keepdims=True) - 1.0
    eot_ref[:] = eot.astype(jnp.int32)


_router = pl.pallas_call(
    _router_kernel,
    out_shape=[
        jax.ShapeDtypeStruct((_N, _E), jnp.float32),
        jax.ShapeDtypeStruct((1, 1), jnp.float32),
        jax.ShapeDtypeStruct((_N, 16), jnp.float32),
        jax.ShapeDtypeStruct((_N, 16), jnp.float32),
        jax.ShapeDtypeStruct((_N, 1), jnp.int32),
        jax.ShapeDtypeStruct((_N, 1), jnp.int32),
        jax.ShapeDtypeStruct((_G, 1), jnp.int32),
    ],
)


@functools.cache
def _sc_mesh():
    return plsc.VectorSubcoreMesh(core_axis_name="c", subcore_axis_name="s")


def _dispatch_body(x_hbm, pa_hbm, pb_hbm, xs_hbm,
                   pa_v, pb_v, rows_v, sem1, sem2):
    wid = lax.axis_index("s") * _NC + lax.axis_index("c")
    bt = wid * _CT
    pltpu.sync_copy(pa_hbm.at[pl.ds(bt, _CT)], pa_v)
    pltpu.sync_copy(pb_hbm.at[pl.ds(bt, _CT)], pb_v)
    pltpu.sync_copy(x_hbm.at[pl.ds(bt, _CT)], rows_v)
    c1 = pltpu.async_copy(rows_v, xs_hbm.at[pa_v], sem1)
    c2 = pltpu.async_copy(rows_v, xs_hbm.at[pb_v], sem2)
    c1.wait()
    c2.wait()


@functools.cache
def _dispatch():
    return pl.kernel(
        _dispatch_body,
        out_type=jax.ShapeDtypeStruct((_P, _DIN), jnp.float32),
        mesh=_sc_mesh(),
        scratch_types=[
            pltpu.VMEM((_CT,), jnp.int32),
            pltpu.VMEM((_CT,), jnp.int32),
            pltpu.VMEM((_CT, _DIN), jnp.float32),
            pltpu.SemaphoreType.DMA,
            pltpu.SemaphoreType.DMA,
        ],
    )


def _ffn_kernel(eot_ref, xs_ref, w1_ref, b1_ref, w2_ref, b2_ref,
                pa_ref, pb_ref, twa_ref, twb_ref, o_ref, pairs_scr):
    t = pl.program_id(0)

    @pl.when(t < _G)
    def _ffn_step():
        xt = xs_ref[:]
        xt = jnp.where(jnp.abs(xt) <= 3.0e38, xt, 0.0)
        h = jnp.dot(xt, w1_ref[0], preferred_element_type=jnp.float32)
        h = jnp.maximum(h + b1_ref[0], 0.0)
        o = jnp.dot(h, w2_ref[0], preferred_element_type=jnp.float32)
        pairs_scr[pl.ds(t * _TM, _TM), :] = (o + b2_ref[0]).astype(jnp.bfloat16)

    @pl.when(t >= _G)
    def _combine_step():
        b = t - _G
        rows = pl.ds(b * _TM, _TM)
        pa = pa_ref[rows, :]
        pb = pb_ref[rows, :]
        wa = twa_ref[rows, 0:1]
        wb = twb_ref[rows, 0:1]
        j = lax.broadcasted_iota(jnp.int32, (_TM, _P), 1)
        g = (jnp.where(j == pa, wa, 0.0)
             + jnp.where(j == pb, wb, 0.0)).astype(jnp.bfloat16)
        o_ref[:] = jnp.dot(g, pairs_scr[:],
                           preferred_element_type=jnp.float32)


_ffn = pl.pallas_call(
    _ffn_kernel,
    grid_spec=pltpu.PrefetchScalarGridSpec(
        num_scalar_prefetch=1,
        grid=(_G + _NB,),
        in_specs=[
            pl.BlockSpec(
                (_TM, _DIN),
                lambda t, eot: (jnp.minimum(t, _G - 1), 0)),
            pl.BlockSpec(
                (1, _DIN, _DH),
                lambda t, eot: (eot[jnp.minimum(t, _G - 1)], 0, 0)),
            pl.BlockSpec(
                (1, 1, _DH),
                lambda t, eot: (eot[jnp.minimum(t, _G - 1)], 0, 0)),
            pl.BlockSpec(
                (1, _DH, _DOUT),
                lambda t, eot: (eot[jnp.minimum(t, _G - 1)], 0, 0)),
            pl.BlockSpec(
                (1, 1, _DOUT),
                lambda t, eot: (eot[jnp.minimum(t, _G - 1)], 0, 0)),
            pl.BlockSpec((_N, 1), lambda t, eot: (0, 0)),
            pl.BlockSpec((_N, 1), lambda t, eot: (0, 0)),
            pl.BlockSpec((_N, 16), lambda t, eot: (0, 0)),
            pl.BlockSpec((_N, 16), lambda t, eot: (0, 0)),
        ],
        out_specs=pl.BlockSpec(
            (_TM, _DOUT), lambda t, eot: (jnp.maximum(t - _G, 0), 0)),
        scratch_shapes=[pltpu.VMEM((_P, _DOUT), jnp.bfloat16)],
    ),
    out_shape=jax.ShapeDtypeStruct((_N, _DOUT), jnp.float32),
    compiler_params=pltpu.CompilerParams(
        dimension_semantics=("arbitrary",)),
)


def kernel(x, gate_w, gate_b, W1, b1, W2, b2):
    probs, aux, twa, twb, pa2, pb2, eot2 = _router(
        x, gate_w, gate_b.reshape(1, _E))
    pos_a = pa2.reshape(_N)
    pos_b = pb2.reshape(_N)
    eot = eot2.reshape(_G)
    xs = _dispatch()(x, pos_a, pos_b)
    out = _ffn(eot, xs, W1, b1.reshape(_E, 1, _DH),
               W2, b2.reshape(_E, 1, _DOUT), pa2, pb2, twa, twb)
    return out, aux.reshape(()), probs


# speedup vs baseline: 1.2493x; 1.0698x over previous
import functools

import jax
import jax.numpy as jnp
from jax import lax
from jax.experimental import pallas as pl
from jax.experimental.pallas import tpu as pltpu
from jax.experimental.pallas import tpu_sc as plsc

_E = 8
_K = 2
_DIN = 768
_DH = 3072
_DOUT = 768
_N = 2048

_TM = 128
_NPAIR = _N * _K
_P = 5120
_G = _P // _TM
_NB = _N // _TM
_GP = 64

_NC = 2
_NS = 16
_NW = _NC * _NS
_CT = _N // _NW


def _router_kernel(x_ref, gw_ref, gb_ref,
                   probs_ref, aux_ref, twa_ref, twb_ref,
                   pa_ref, pb_ref, sched_ref):
    x = x_ref[:]
    logits = jnp.dot(x, gw_ref[:], preferred_element_type=jnp.float32) + gb_ref[:]
    m = jnp.max(logits, axis=1, keepdims=True)
    ex = jnp.exp(logits - m)
    probs = ex / jnp.sum(ex, axis=1, keepdims=True)
    probs_ref[:] = probs

    mp = jnp.mean(probs, axis=0, keepdims=True)
    aux_ref[:] = jnp.sum(mp * jnp.log(mp * _E + 1e-10), axis=1, keepdims=True)

    ii = lax.broadcasted_iota(jnp.int32, (_N, _E), 1)
    v1 = jnp.max(probs, axis=1, keepdims=True)
    i1 = jnp.min(jnp.where(probs >= v1, ii, _E), axis=1, keepdims=True)
    oh1 = ii == i1
    pm = jnp.where(oh1, -1.0, probs)
    v2 = jnp.max(pm, axis=1, keepdims=True)
    i2 = jnp.min(jnp.where(pm >= v2, ii, _E), axis=1, keepdims=True)
    oh2 = ii == i2
    den = v1 + v2 + 1e-10
    twa_ref[:] = jnp.broadcast_to(v1 / den, (_N, 16))
    twb_ref[:] = jnp.broadcast_to(v2 / den, (_N, 16))

    cnt = oh1.astype(jnp.float32) + oh2.astype(jnp.float32)
    nb = _N // 128
    r = lax.broadcasted_iota(jnp.int32, (128, 128), 0)
    c = lax.broadcasted_iota(jnp.int32, (128, 128), 1)
    tril = (c < r).astype(jnp.float32)
    blocks, sums = [], []
    for b in range(nb):
        blk = cnt[b * 128:(b + 1) * 128, :]
        blocks.append(jnp.dot(tril, blk, preferred_element_type=jnp.float32))
        sums.append(jnp.sum(blk, axis=0, keepdims=True))
    s = jnp.concatenate(sums, axis=0)
    r2 = lax.broadcasted_iota(jnp.int32, (nb, nb), 0)
    c2 = lax.broadcasted_iota(jnp.int32, (nb, nb), 1)
    tril2 = (c2 < r2).astype(jnp.float32)
    carry = jnp.dot(tril2, s, preferred_element_type=jnp.float32)
    cex = jnp.concatenate(
        [blocks[b] + carry[b:b + 1, :] for b in range(nb)], axis=0)

    tot = jnp.sum(s, axis=0, keepdims=True)
    cpad = (tot.astype(jnp.int32) + (_TM - 1)) // _TM * _TM
    r3 = lax.broadcasted_iota(jnp.int32, (_E, _E), 0)
    c3 = lax.broadcasted_iota(jnp.int32, (_E, _E), 1)
    sup = (r3 < c3).astype(jnp.float32)
    off = jnp.dot(cpad.astype(jnp.float32), sup,
                  preferred_element_type=jnp.float32)

    base = off + cex
    pa = jnp.sum(jnp.where(oh1, base, 0.0), axis=1, keepdims=True)
    pb = jnp.sum(jnp.where(oh2, base, 0.0), axis=1, keepdims=True)
    pa_ref[:] = pa.astype(jnp.int32)
    pb_ref[:] = pb.astype(jnp.int32)

    tv = (lax.broadcasted_iota(jnp.int32, (_G, _E), 0) * _TM).astype(jnp.float32)
    eot = jnp.sum((off <= tv).astype(jnp.float32), axis=1, keepdims=True) - 1.0

    prev = jnp.concatenate(
        [jnp.full((1, 1), -1.0, jnp.float32), eot[:_G - 1, :]], axis=0)
    is_sw = (eot != prev).astype(jnp.float32)
    rg = lax.broadcasted_iota(jnp.int32, (_G, _G), 0)
    cg = lax.broadcasted_iota(jnp.int32, (_G, _G), 1)
    linc = (cg <= rg).astype(jnp.float32)
    c = jnp.dot(linc, is_sw, preferred_element_type=jnp.float32)
    slot = ((c.astype(jnp.int32) - 1) & 1).astype(jnp.float32)
    kk = (lax.broadcasted_iota(jnp.int32, (1, _E), 1) + 1).astype(
        jnp.float32)
    h_runs = is_sw * (c == kk).astype(jnp.float32)
    dn = (((0,), (0,)), ((), ()))
    run_expert = lax.dot_general(h_runs, eot, dn,
                                 preferred_element_type=jnp.float32)
    run_valid = lax.dot_general(h_runs, jnp.ones((_G, 1), jnp.float32), dn,
                                preferred_element_type=jnp.float32)
    oh_next = (c + 1.0 == kk).astype(jnp.float32)
    nxt_f = jnp.dot(oh_next, run_expert, preferred_element_type=jnp.float32)
    rvg = jnp.dot(oh_next, run_valid, preferred_element_type=jnp.float32)
    nxt = jnp.where(rvg > 0.0, nxt_f, -1.0)

    zpad = jnp.zeros((_GP - _G, 1), jnp.float32)
    cols = [jnp.concatenate([v, zpad], axis=0)
            for v in (is_sw, slot, nxt, eot)]
    cols.append(jnp.zeros((_GP, _E - 4), jnp.float32))
    sched_ref[:] = jnp.concatenate(cols, axis=1).astype(jnp.int32)


_router = pl.pallas_call(
    _router_kernel,
    out_shape=[
        jax.ShapeDtypeStruct((_N, _E), jnp.float32),
        jax.ShapeDtypeStruct((1, 1), jnp.float32),
        jax.ShapeDtypeStruct((_N, 16), jnp.float32),
        jax.ShapeDtypeStruct((_N, 16), jnp.float32),
        jax.ShapeDtypeStruct((_N, 1), jnp.int32),
        jax.ShapeDtypeStruct((_N, 1), jnp.int32),
        jax.ShapeDtypeStruct((_GP, _E), jnp.int32),
    ],
)


@functools.cache
def _sc_mesh():
    return plsc.VectorSubcoreMesh(core_axis_name="c", subcore_axis_name="s")


def _dispatch_body(x_hbm, pa_hbm, pb_hbm, xs_hbm,
                   pa_v, pb_v, rows_v, sem1, sem2):
    wid = lax.axis_index("s") * _NC + lax.axis_index("c")
    bt = wid * _CT
    pltpu.sync_copy(pa_hbm.at[pl.ds(bt, _CT)], pa_v)
    pltpu.sync_copy(pb_hbm.at[pl.ds(bt, _CT)], pb_v)
    pltpu.sync_copy(x_hbm.at[pl.ds(bt, _CT)], rows_v)
    c1 = pltpu.async_copy(rows_v, xs_hbm.at[pa_v], sem1)
    c2 = pltpu.async_copy(rows_v, xs_hbm.at[pb_v], sem2)
    c1.wait()
    c2.wait()


@functools.cache
def _dispatch():
    return pl.kernel(
        _dispatch_body,
        out_type=jax.ShapeDtypeStruct((_P, _DIN), jnp.float32),
        mesh=_sc_mesh(),
        scratch_types=[
            pltpu.VMEM((_CT,), jnp.int32),
            pltpu.VMEM((_CT,), jnp.int32),
            pltpu.VMEM((_CT, _DIN), jnp.float32),
            pltpu.SemaphoreType.DMA,
            pltpu.SemaphoreType.DMA,
        ],
    )


def _ffn_kernel(sched_ref, xs_ref, b1_ref, b2_ref,
                pa_ref, pb_ref, twa_ref, twb_ref, w1_hbm, w2_hbm,
                o_ref, pairs_scr, w1_scr, w2_scr, s1a, s1b, s2a, s2b):
    t = pl.program_id(0)
    is_sw = sched_ref[t, 0]
    slot = sched_ref[t, 1]
    nxt = sched_ref[t, 2]
    cur = sched_ref[t, 3]

    @pl.when(t == 0)
    def _prime():
        pltpu.make_async_copy(
            w1_hbm.at[pl.ds(cur, 1)], w1_scr.at[pl.ds(0, 1)], s1a).start()
        pltpu.make_async_copy(
            w2_hbm.at[pl.ds(cur, 1)], w2_scr.at[pl.ds(0, 1)], s2a).start()

    @pl.when(jnp.logical_and(is_sw == 1, slot == 0))
    def _wait_a():
        pltpu.make_async_copy(
            w1_hbm.at[pl.ds(0, 1)], w1_scr.at[pl.ds(0, 1)], s1a).wait()
        pltpu.make_async_copy(
            w2_hbm.at[pl.ds(0, 1)], w2_scr.at[pl.ds(0, 1)], s2a).wait()

    @pl.when(jnp.logical_and(is_sw == 1, slot == 1))
    def _wait_b():
        pltpu.make_async_copy(
            w1_hbm.at[pl.ds(0, 1)], w1_scr.at[pl.ds(1, 1)], s1b).wait()
        pltpu.make_async_copy(
            w2_hbm.at[pl.ds(0, 1)], w2_scr.at[pl.ds(1, 1)], s2b).wait()

    fetch_next = jnp.logical_and(is_sw == 1, nxt >= 0)

    @pl.when(jnp.logical_and(fetch_next, slot == 0))
    def _fetch_b():
        pltpu.make_async_copy(
            w1_hbm.at[pl.ds(nxt, 1)], w1_scr.at[pl.ds(1, 1)], s1b).start()
        pltpu.make_async_copy(
            w2_hbm.at[pl.ds(nxt, 1)], w2_scr.at[pl.ds(1, 1)], s2b).start()

    @pl.when(jnp.logical_and(fetch_next, slot == 1))
    def _fetch_a():
        pltpu.make_async_copy(
            w1_hbm.at[pl.ds(nxt, 1)], w1_scr.at[pl.ds(0, 1)], s1a).start()
        pltpu.make_async_copy(
            w2_hbm.at[pl.ds(nxt, 1)], w2_scr.at[pl.ds(0, 1)], s2a).start()

    @pl.when(t < _G)
    def _ffn_step():
        xt = xs_ref[:]
        xt = jnp.where(jnp.abs(xt) <= 3.0e38, xt, 0.0)
        h = jnp.dot(xt, w1_scr[slot], preferred_element_type=jnp.float32)
        h = jnp.maximum(h + b1_ref[cur], 0.0)
        o = jnp.dot(h, w2_scr[slot], preferred_element_type=jnp.float32)
        pairs_scr[pl.ds(t * _TM, _TM), :] = (o + b2_ref[cur]).astype(
            jnp.bfloat16)

    @pl.when(t >= _G)
    def _combine_step():
        b = t - _G
        rows = pl.ds(b * _TM, _TM)
        pa = pa_ref[rows, :]
        pb = pb_ref[rows, :]
        wa = twa_ref[rows, 0:1]
        wb = twb_ref[rows, 0:1]
        j = lax.broadcasted_iota(jnp.int32, (_TM, _P), 1)
        g = (jnp.where(j == pa, wa, 0.0)
             + jnp.where(j == pb, wb, 0.0)).astype(jnp.bfloat16)
        o_ref[:] = jnp.dot(g, pairs_scr[:],
                           preferred_element_type=jnp.float32)


_ffn = pl.pallas_call(
    _ffn_kernel,
    grid_spec=pltpu.PrefetchScalarGridSpec(
        num_scalar_prefetch=1,
        grid=(_G + _NB,),
        in_specs=[
            pl.BlockSpec(
                (_TM, _DIN),
                lambda t, sched: (jnp.minimum(t, _G - 1), 0)),
            pl.BlockSpec((_E, 1, _DH), lambda t, sched: (0, 0, 0)),
            pl.BlockSpec((_E, 1, _DOUT), lambda t, sched: (0, 0, 0)),
            pl.BlockSpec((_N, 1), lambda t, sched: (0, 0)),
            pl.BlockSpec((_N, 1), lambda t, sched: (0, 0)),
            pl.BlockSpec((_N, 16), lambda t, sched: (0, 0)),
            pl.BlockSpec((_N, 16), lambda t, sched: (0, 0)),
            pl.BlockSpec(memory_space=pl.ANY),
            pl.BlockSpec(memory_space=pl.ANY),
        ],
        out_specs=pl.BlockSpec(
            (_TM, _DOUT), lambda t, sched: (jnp.maximum(t - _G, 0), 0)),
        scratch_shapes=[
            pltpu.VMEM((_P, _DOUT), jnp.bfloat16),
            pltpu.VMEM((2, _DIN, _DH), jnp.float32),
            pltpu.VMEM((2, _DH, _DOUT), jnp.float32),
            pltpu.SemaphoreType.DMA,
            pltpu.SemaphoreType.DMA,
            pltpu.SemaphoreType.DMA,
            pltpu.SemaphoreType.DMA,
        ],
    ),
    out_shape=jax.ShapeDtypeStruct((_N, _DOUT), jnp.float32),
    compiler_params=pltpu.CompilerParams(
        dimension_semantics=("arbitrary",)),
)


def kernel(x, gate_w, gate_b, W1, b1, W2, b2):
    probs, aux, twa, twb, pa2, pb2, sched = _router(
        x, gate_w, gate_b.reshape(1, _E))
    pos_a = pa2.reshape(_N)
    pos_b = pb2.reshape(_N)
    xs = _dispatch()(x, pos_a, pos_b)
    out = _ffn(sched, xs, b1.reshape(_E, 1, _DH), b2.reshape(_E, 1, _DOUT),
               pa2, pb2, twa, twb, W1, W2)
    return out, aux.reshape(()), probs


# speedup vs baseline: 1.2508x; 1.0012x over previous
import functools

import jax
import jax.numpy as jnp
from jax import lax
from jax.experimental import pallas as pl
from jax.experimental.pallas import tpu as pltpu
from jax.experimental.pallas import tpu_sc as plsc

_E = 8
_K = 2
_DIN = 768
_DH = 3072
_DOUT = 768
_N = 2048

_TM = 128
_NPAIR = _N * _K
_P = 5120
_G = _P // _TM
_NB = _N // _TM
_GP = 64

_NC = 2
_NS = 16
_NW = _NC * _NS
_CT = _N // _NW


def _router_kernel(x_ref, gw_ref, gb_ref,
                   probs_ref, aux_ref, twa_ref, twb_ref,
                   pa_ref, pb_ref, sched_ref):
    x = x_ref[:]
    logits = jnp.dot(x, gw_ref[:], preferred_element_type=jnp.float32) + gb_ref[:]
    m = jnp.max(logits, axis=1, keepdims=True)
    ex = jnp.exp(logits - m)
    probs = ex / jnp.sum(ex, axis=1, keepdims=True)
    probs_ref[:] = probs

    mp = jnp.mean(probs, axis=0, keepdims=True)
    aux_ref[:] = jnp.sum(mp * jnp.log(mp * _E + 1e-10), axis=1, keepdims=True)

    ii = lax.broadcasted_iota(jnp.int32, (_N, _E), 1)
    v1 = jnp.max(probs, axis=1, keepdims=True)
    i1 = jnp.min(jnp.where(probs >= v1, ii, _E), axis=1, keepdims=True)
    oh1 = ii == i1
    pm = jnp.where(oh1, -1.0, probs)
    v2 = jnp.max(pm, axis=1, keepdims=True)
    i2 = jnp.min(jnp.where(pm >= v2, ii, _E), axis=1, keepdims=True)
    oh2 = ii == i2
    den = v1 + v2 + 1e-10
    twa_ref[:] = jnp.broadcast_to(v1 / den, (_N, 16))
    twb_ref[:] = jnp.broadcast_to(v2 / den, (_N, 16))

    cnt = oh1.astype(jnp.float32) + oh2.astype(jnp.float32)
    nb = _N // 128
    r = lax.broadcasted_iota(jnp.int32, (128, 128), 0)
    c = lax.broadcasted_iota(jnp.int32, (128, 128), 1)
    tril = (c < r).astype(jnp.float32)
    blocks, sums = [], []
    for b in range(nb):
        blk = cnt[b * 128:(b + 1) * 128, :]
        blocks.append(jnp.dot(tril, blk, preferred_element_type=jnp.float32))
        sums.append(jnp.sum(blk, axis=0, keepdims=True))
    s = jnp.concatenate(sums, axis=0)
    r2 = lax.broadcasted_iota(jnp.int32, (nb, nb), 0)
    c2 = lax.broadcasted_iota(jnp.int32, (nb, nb), 1)
    tril2 = (c2 < r2).astype(jnp.float32)
    carry = jnp.dot(tril2, s, preferred_element_type=jnp.float32)
    cex = jnp.concatenate(
        [blocks[b] + carry[b:b + 1, :] for b in range(nb)], axis=0)

    tot = jnp.sum(s, axis=0, keepdims=True)
    cpad = (tot.astype(jnp.int32) + (_TM - 1)) // _TM * _TM
    r3 = lax.broadcasted_iota(jnp.int32, (_E, _E), 0)
    c3 = lax.broadcasted_iota(jnp.int32, (_E, _E), 1)
    sup = (r3 < c3).astype(jnp.float32)
    off = jnp.dot(cpad.astype(jnp.float32), sup,
                  preferred_element_type=jnp.float32)

    base = off + cex
    pa = jnp.sum(jnp.where(oh1, base, 0.0), axis=1, keepdims=True)
    pb = jnp.sum(jnp.where(oh2, base, 0.0), axis=1, keepdims=True)
    pa_ref[:] = pa.astype(jnp.int32)
    pb_ref[:] = pb.astype(jnp.int32)

    tv = (lax.broadcasted_iota(jnp.int32, (_G, _E), 0) * _TM).astype(jnp.float32)
    eot = jnp.sum((off <= tv).astype(jnp.float32), axis=1, keepdims=True) - 1.0

    prev = jnp.concatenate(
        [jnp.full((1, 1), -1.0, jnp.float32), eot[:_G - 1, :]], axis=0)
    is_sw = (eot != prev).astype(jnp.float32)
    rg = lax.broadcasted_iota(jnp.int32, (_G, _G), 0)
    cg = lax.broadcasted_iota(jnp.int32, (_G, _G), 1)
    linc = (cg <= rg).astype(jnp.float32)
    c = jnp.dot(linc, is_sw, preferred_element_type=jnp.float32)
    slot = ((c.astype(jnp.int32) - 1) & 1).astype(jnp.float32)
    kk = (lax.broadcasted_iota(jnp.int32, (1, _E), 1) + 1).astype(
        jnp.float32)
    h_runs = is_sw * (c == kk).astype(jnp.float32)
    dn = (((0,), (0,)), ((), ()))
    run_expert = lax.dot_general(h_runs, eot, dn,
                                 preferred_element_type=jnp.float32)
    run_valid = lax.dot_general(h_runs, jnp.ones((_G, 1), jnp.float32), dn,
                                preferred_element_type=jnp.float32)
    oh_next = (c + 1.0 == kk).astype(jnp.float32)
    nxt_f = jnp.dot(oh_next, run_expert, preferred_element_type=jnp.float32)
    rvg = jnp.dot(oh_next, run_valid, preferred_element_type=jnp.float32)
    nxt = jnp.where(rvg > 0.0, nxt_f, -1.0)

    zpad = jnp.zeros((_GP - _G, 1), jnp.float32)
    cols = [jnp.concatenate([v, zpad], axis=0)
            for v in (is_sw, slot, nxt, eot)]
    cols.append(jnp.zeros((_GP, _E - 4), jnp.float32))
    sched_ref[:] = jnp.concatenate(cols, axis=1).astype(jnp.int32)


_router = pl.pallas_call(
    _router_kernel,
    out_shape=[
        jax.ShapeDtypeStruct((_N, _E), jnp.float32),
        jax.ShapeDtypeStruct((1, 1), jnp.float32),
        jax.ShapeDtypeStruct((_N, 16), jnp.float32),
        jax.ShapeDtypeStruct((_N, 16), jnp.float32),
        jax.ShapeDtypeStruct((_N, 1), jnp.int32),
        jax.ShapeDtypeStruct((_N, 1), jnp.int32),
        jax.ShapeDtypeStruct((_GP, _E), jnp.int32),
    ],
)


@functools.cache
def _sc_mesh():
    return plsc.VectorSubcoreMesh(core_axis_name="c", subcore_axis_name="s")


def _dispatch_body(x_hbm, pa_hbm, pb_hbm, xs_hbm,
                   pa_v, pb_v, rows_v, sem1, sem2):
    wid = lax.axis_index("s") * _NC + lax.axis_index("c")
    bt = wid * _CT
    pltpu.sync_copy(pa_hbm.at[pl.ds(bt, _CT)], pa_v)
    pltpu.sync_copy(pb_hbm.at[pl.ds(bt, _CT)], pb_v)
    pltpu.sync_copy(x_hbm.at[pl.ds(bt, _CT)], rows_v)
    c1 = pltpu.async_copy(rows_v, xs_hbm.at[pa_v], sem1)
    c2 = pltpu.async_copy(rows_v, xs_hbm.at[pb_v], sem2)
    c1.wait()
    c2.wait()


@functools.cache
def _dispatch():
    return pl.kernel(
        _dispatch_body,
        out_type=jax.ShapeDtypeStruct((_P, _DIN), jnp.float32),
        mesh=_sc_mesh(),
        scratch_types=[
            pltpu.VMEM((_CT,), jnp.int32),
            pltpu.VMEM((_CT,), jnp.int32),
            pltpu.VMEM((_CT, _DIN), jnp.float32),
            pltpu.SemaphoreType.DMA,
            pltpu.SemaphoreType.DMA,
        ],
    )


def _ffn_kernel(sched_ref, xs_ref, b1_ref, b2_ref,
                pa_ref, pb_ref, twa_ref, twb_ref, w1_hbm, w2_hbm,
                o_ref, pairs_scr, w1_scr, w2_scr, s1a, s1b, s2a, s2b):
    t = pl.program_id(0)
    is_sw = sched_ref[t, 0]
    slot = sched_ref[t, 1]
    nxt = sched_ref[t, 2]
    cur = sched_ref[t, 3]

    @pl.when(t == 0)
    def _prime():
        pltpu.make_async_copy(
            w1_hbm.at[pl.ds(cur, 1)], w1_scr.at[pl.ds(0, 1)], s1a).start()
        pltpu.make_async_copy(
            w2_hbm.at[pl.ds(cur, 1)], w2_scr.at[pl.ds(0, 1)], s2a).start()

    @pl.when(jnp.logical_and(is_sw == 1, slot == 0))
    def _wait_a():
        pltpu.make_async_copy(
            w1_hbm.at[pl.ds(0, 1)], w1_scr.at[pl.ds(0, 1)], s1a).wait()
        pltpu.make_async_copy(
            w2_hbm.at[pl.ds(0, 1)], w2_scr.at[pl.ds(0, 1)], s2a).wait()

    @pl.when(jnp.logical_and(is_sw == 1, slot == 1))
    def _wait_b():
        pltpu.make_async_copy(
            w1_hbm.at[pl.ds(0, 1)], w1_scr.at[pl.ds(1, 1)], s1b).wait()
        pltpu.make_async_copy(
            w2_hbm.at[pl.ds(0, 1)], w2_scr.at[pl.ds(1, 1)], s2b).wait()

    fetch_next = jnp.logical_and(is_sw == 1, nxt >= 0)

    @pl.when(jnp.logical_and(fetch_next, slot == 0))
    def _fetch_b():
        pltpu.make_async_copy(
            w1_hbm.at[pl.ds(nxt, 1)], w1_scr.at[pl.ds(1, 1)], s1b).start()
        pltpu.make_async_copy(
            w2_hbm.at[pl.ds(nxt, 1)], w2_scr.at[pl.ds(1, 1)], s2b).start()

    @pl.when(jnp.logical_and(fetch_next, slot == 1))
    def _fetch_a():
        pltpu.make_async_copy(
            w1_hbm.at[pl.ds(nxt, 1)], w1_scr.at[pl.ds(0, 1)], s1a).start()
        pltpu.make_async_copy(
            w2_hbm.at[pl.ds(nxt, 1)], w2_scr.at[pl.ds(0, 1)], s2a).start()

    def _ffn_with(sl):
        xt = xs_ref[:]
        xt = jnp.where(jnp.abs(xt) <= 3.0e38, xt, 0.0)
        h = jnp.dot(xt, w1_scr[sl], preferred_element_type=jnp.float32)
        h = jnp.maximum(h + b1_ref[cur], 0.0)
        o = jnp.dot(h, w2_scr[sl], preferred_element_type=jnp.float32)
        pairs_scr[pl.ds(t * _TM, _TM), :] = (o + b2_ref[cur]).astype(
            jnp.bfloat16)

    @pl.when(jnp.logical_and(t < _G, slot == 0))
    def _ffn_step_a():
        _ffn_with(0)

    @pl.when(jnp.logical_and(t < _G, slot == 1))
    def _ffn_step_b():
        _ffn_with(1)

    @pl.when(t >= _G)
    def _combine_step():
        b = t - _G
        rows = pl.ds(b * _TM, _TM)
        pa = pa_ref[rows, :]
        pb = pb_ref[rows, :]
        wa = twa_ref[rows, 0:1]
        wb = twb_ref[rows, 0:1]
        j = lax.broadcasted_iota(jnp.int32, (_TM, _P), 1)
        g = (jnp.where(j == pa, wa, 0.0)
             + jnp.where(j == pb, wb, 0.0)).astype(jnp.bfloat16)
        o_ref[:] = jnp.dot(g, pairs_scr[:],
                           preferred_element_type=jnp.float32)


_ffn = pl.pallas_call(
    _ffn_kernel,
    grid_spec=pltpu.PrefetchScalarGridSpec(
        num_scalar_prefetch=1,
        grid=(_G + _NB,),
        in_specs=[
            pl.BlockSpec(
                (_TM, _DIN),
                lambda t, sched: (jnp.minimum(t, _G - 1), 0)),
            pl.BlockSpec((_E, 1, _DH), lambda t, sched: (0, 0, 0)),
            pl.BlockSpec((_E, 1, _DOUT), lambda t, sched: (0, 0, 0)),
            pl.BlockSpec((_N, 1), lambda t, sched: (0, 0)),
            pl.BlockSpec((_N, 1), lambda t, sched: (0, 0)),
            pl.BlockSpec((_N, 16), lambda t, sched: (0, 0)),
            pl.BlockSpec((_N, 16), lambda t, sched: (0, 0)),
            pl.BlockSpec(memory_space=pl.ANY),
            pl.BlockSpec(memory_space=pl.ANY),
        ],
        out_specs=pl.BlockSpec(
            (_TM, _DOUT), lambda t, sched: (jnp.maximum(t - _G, 0), 0)),
        scratch_shapes=[
            pltpu.VMEM((_P, _DOUT), jnp.bfloat16),
            pltpu.VMEM((2, _DIN, _DH), jnp.float32),
            pltpu.VMEM((2, _DH, _DOUT), jnp.float32),
            pltpu.SemaphoreType.DMA,
            pltpu.SemaphoreType.DMA,
            pltpu.SemaphoreType.DMA,
            pltpu.SemaphoreType.DMA,
        ],
    ),
    out_shape=jax.ShapeDtypeStruct((_N, _DOUT), jnp.float32),
    compiler_params=pltpu.CompilerParams(
        dimension_semantics=("arbitrary",)),
)


def kernel(x, gate_w, gate_b, W1, b1, W2, b2):
    probs, aux, twa, twb, pa2, pb2, sched = _router(
        x, gate_w, gate_b.reshape(1, _E))
    pos_a = pa2.reshape(_N)
    pos_b = pb2.reshape(_N)
    xs = _dispatch()(x, pos_a, pos_b)
    out = _ffn(sched, xs, b1.reshape(_E, 1, _DH), b2.reshape(_E, 1, _DOUT),
               pa2, pb2, twa, twb, W1, W2)
    return out, aux.reshape(()), probs
